# speedup vs baseline: 1.1366x; 1.1366x over previous
import functools

import jax
import jax.numpy as jnp
from jax.experimental import pallas as pl
from jax.experimental.pallas import tpu as pltpu

_N = 10000
_E = 320000
_IN = 128
_OUT = 128
_H1 = 8
_C1 = 8

_ROWS = 1000


def _dense_body(x_ref, w_ref, asrc_ref, adst_ref, h_ref, es_ref, ed_ref):
    h = jnp.dot(x_ref[...], w_ref[...], preferred_element_type=jnp.float32)
    h_ref[...] = h
    es_ref[...] = jnp.dot(h, asrc_ref[...], preferred_element_type=jnp.float32)
    ed_ref[...] = jnp.dot(h, adst_ref[...], preferred_element_type=jnp.float32)


def _dense_stage(x, W, a_src, a_dst, heads, ch):
    n, k = x.shape[0], W.shape[1]
    eye = jnp.eye(heads, dtype=jnp.float32)
    Asrc = (a_src.reshape(heads, ch)[:, :, None] * eye[:, None, :]).reshape(k, heads)
    Adst = (a_dst.reshape(heads, ch)[:, :, None] * eye[:, None, :]).reshape(k, heads)
    grid = n // _ROWS
    h, es, ed = pl.pallas_call(
        _dense_body,
        grid=(grid,),
        in_specs=[
            pl.BlockSpec((_ROWS, x.shape[1]), lambda i: (i, 0)),
            pl.BlockSpec((x.shape[1], k), lambda i: (0, 0)),
            pl.BlockSpec((k, heads), lambda i: (0, 0)),
            pl.BlockSpec((k, heads), lambda i: (0, 0)),
        ],
        out_specs=[
            pl.BlockSpec((_ROWS, k), lambda i: (i, 0)),
            pl.BlockSpec((_ROWS, heads), lambda i: (i, 0)),
            pl.BlockSpec((_ROWS, heads), lambda i: (i, 0)),
        ],
        out_shape=[
            jax.ShapeDtypeStruct((n, k), jnp.float32),
            jax.ShapeDtypeStruct((n, heads), jnp.float32),
            jax.ShapeDtypeStruct((n, heads), jnp.float32),
        ],
    )(x, W, Asrc, Adst)
    return h, es, ed


def _edge_phase(h, es, ed, src, dst, heads, ch, n):
    e = es[src] + ed[dst]
    e = jax.nn.leaky_relu(e, 0.2)
    m = jnp.max(e)
    p = jnp.exp(e - m)
    denom = jax.ops.segment_sum(p, dst, num_segments=n)
    hh = h.reshape(n, heads, ch)
    msg = hh[src] * p[..., None]
    out = jax.ops.segment_sum(msg, dst, num_segments=n)
    return out / (denom[..., None] + 1e-16)


def kernel(x, edge_index, W1, att_src1, att_dst1, b1, W2, att_src2, att_dst2, b2):
    loop = jnp.arange(_N, dtype=edge_index.dtype)
    src = jnp.concatenate([edge_index[0], loop])
    dst = jnp.concatenate([edge_index[1], loop])

    h1, es1, ed1 = _dense_stage(x, W1, att_src1, att_dst1, _H1, _C1)
    o1 = _edge_phase(h1, es1, ed1, src, dst, _H1, _C1, _N)
    o1 = jax.nn.relu(o1.reshape(_N, _H1 * _C1) + b1)

    h2, es2, ed2 = _dense_stage(o1, W2, att_src2, att_dst2, 1, _OUT)
    o2 = _edge_phase(h2, es2, ed2, src, dst, 1, _OUT, _N)
    o2 = o2.reshape(_N, _OUT) + b2
    return jax.nn.log_softmax(o2, axis=1)


# speedup vs baseline: 19.9528x; 17.5554x over previous
import functools

import jax
import jax.numpy as jnp
from jax import lax
from jax.experimental import pallas as pl
from jax.experimental.pallas import tpu as pltpu
from jax.experimental.pallas import tpu_sc as plsc

_N = 10000
_E = 320000
_IN = 128
_OUT = 128
_H1 = 8
_C1 = 8

_ROWS = 1000

_NC, _NS, _L = 2, 16, 16
_B = 128
_NCHUNK = 162
_ETP = _NS * _NCHUNK * _B
_NP = 10240
_RPT = _NP // _NS
_ZR = 32
_NED = _N + 8
_HP = 8


def _dense1_body(x_ref, w_ref, asrc_ref, adst_ref, h_ref, es_ref, ed_ref):
    h = jnp.dot(x_ref[...], w_ref[...], preferred_element_type=jnp.float32)
    h_ref[...] = h
    es_ref[...] = jnp.dot(h, asrc_ref[...], preferred_element_type=jnp.float32)
    ed_ref[...] = jnp.dot(h, adst_ref[...], preferred_element_type=jnp.float32)


def _dense2_body(x_ref, b_ref, w_ref, asrc_ref, adst_ref, h_ref, es_ref, ed_ref):
    o = jnp.maximum(x_ref[...] + b_ref[...], 0.0)
    h = jnp.dot(o, w_ref[...], preferred_element_type=jnp.float32)
    h_ref[...] = h
    es_ref[...] = jnp.dot(h, asrc_ref[...], preferred_element_type=jnp.float32)
    ed_ref[...] = jnp.dot(h, adst_ref[...], preferred_element_type=jnp.float32)


def _att_proj(a, heads, ch):
    eye = jnp.eye(heads, dtype=jnp.float32)
    return (a.reshape(heads, ch)[:, :, None] * eye[:, None, :]).reshape(
        heads * ch, heads)


def _dense_stage(x, W, a_src, a_dst, heads, ch, bias=None):
    n, k = x.shape[0], W.shape[1]
    Asrc = _att_proj(a_src, heads, ch)
    Adst = _att_proj(a_dst, heads, ch)
    grid = n // _ROWS
    in_specs = [
        pl.BlockSpec((_ROWS, x.shape[1]), lambda i: (i, 0)),
        pl.BlockSpec((x.shape[1], k), lambda i: (0, 0)),
        pl.BlockSpec((k, heads), lambda i: (0, 0)),
        pl.BlockSpec((k, heads), lambda i: (0, 0)),
    ]
    args = (x, W, Asrc, Adst)
    body = _dense1_body
    if bias is not None:
        in_specs.insert(1, pl.BlockSpec((1, x.shape[1]), lambda i: (0, 0)))
        args = (x, bias.reshape(1, -1), W, Asrc, Adst)
        body = _dense2_body
    return pl.pallas_call(
        body,
        grid=(grid,),
        in_specs=in_specs,
        out_specs=[
            pl.BlockSpec((_ROWS, k), lambda i: (i, 0)),
            pl.BlockSpec((_ROWS, heads), lambda i: (i, 0)),
            pl.BlockSpec((_ROWS, heads), lambda i: (i, 0)),
        ],
        out_shape=[
            jax.ShapeDtypeStruct((n, k), jnp.float32),
            jax.ShapeDtypeStruct((n, heads), jnp.float32),
            jax.ShapeDtypeStruct((n, heads), jnp.float32),
        ],
    )(*args)


def _lsm_body(x_ref, b_ref, o_ref):
    z = x_ref[...] + b_ref[...]
    m = jnp.max(z, axis=1, keepdims=True)
    ez = jnp.exp(z - m)
    lse = jnp.log(jnp.sum(ez, axis=1, keepdims=True))
    o_ref[...] = z - m - lse


def _log_softmax_bias(x, b):
    n, k = x.shape
    return pl.pallas_call(
        _lsm_body,
        grid=(n // _ROWS,),
        in_specs=[
            pl.BlockSpec((_ROWS, k), lambda i: (i, 0)),
            pl.BlockSpec((1, k), lambda i: (0, 0)),
        ],
        out_specs=pl.BlockSpec((_ROWS, k), lambda i: (i, 0)),
        out_shape=jax.ShapeDtypeStruct((n, k), jnp.float32),
    )(x, b.reshape(1, k))


def _make_edge_sc(Hh, C, Fh, n_stripes):
    R = Fh // _L
    log2C = C.bit_length() - 1
    stride = _NP // n_stripes
    srows = _NP if n_stripes == 1 else 5632
    srpt = srows // _NS
    rpt = stride // _NS
    mesh = plsc.VectorSubcoreMesh(core_axis_name="c", subcore_axis_name="s")

    @functools.partial(
        pl.kernel,
        out_type=jax.ShapeDtypeStruct((2, _NP, Fh), jnp.float32),
        mesh=mesh,
        compiler_params=pltpu.CompilerParams(
            needs_layout_passes=False, use_tc_tiling_on_sc=False),
        scratch_types=[
            pltpu.VMEM((_NCHUNK, _B), jnp.int32),
            pltpu.VMEM((_NCHUNK, _B), jnp.int32),
            pltpu.VMEM((_NCHUNK, _B), jnp.int32),
            pltpu.VMEM((_B,), jnp.int32),
            pltpu.VMEM((_B, Fh), jnp.float32),
            pltpu.VMEM((_B, _HP), jnp.float32),
            pltpu.VMEM((_B, _HP), jnp.float32),
            pltpu.VMEM((_B, _HP), jnp.float32),
            pltpu.VMEM((_ZR, Fh), jnp.float32),
            pltpu.VMEM((srpt, _HP), jnp.float32),
            pltpu.VMEM((_ZR, Fh), jnp.float32),
            pltpu.VMEM((_ZR, _HP), jnp.float32),
            pltpu.VMEM((_ZR, Fh), jnp.float32),
            pltpu.VMEM((16,), jnp.float32),
            pltpu.VMEM_SHARED((srows, Fh), jnp.float32),
            pltpu.VMEM_SHARED((srows, _HP), jnp.float32),
            pltpu.SemaphoreType.DMA,
            pltpu.SemaphoreType.DMA,
            pltpu.SemaphoreType.DMA,
        ],
    )
    def k(src_hbm, dst_hbm, h_hbm, es_hbm, ed_hbm, m_hbm, out_hbm,
          src_adj, dst_raw, dst_adj, dstrip, hbuf, esbuf, edbuf, pbuf,
          zbuf, zdbuf, sbuf, dbuf, obuf, mv, S, D, sem_h, sem_e, sem_d):
        c = lax.axis_index("c")
        s = lax.axis_index("s")
        iota = lax.iota(jnp.int32, _L)
        zf = jnp.zeros((_L,), jnp.float32)

        pltpu.sync_copy(src_hbm.at[s], src_adj)
        pltpu.sync_copy(dst_hbm.at[s], dst_raw)
        pltpu.sync_copy(m_hbm, mv)
        m_vec = mv[...]

        cNv = jnp.full((_L,), c * _N, jnp.int32)
        cMv = jnp.full((_L,), c * _NED, jnp.int32)

        def adj_chunk(j, _):
            def adj_vec(q, _):
                sl = pl.ds(q * _L, _L)
                src_adj[j, sl] = src_adj[j, sl] + cNv
                dst_adj[j, sl] = dst_raw[j, sl] + cMv
                return 0
            return lax.fori_loop(0, _B // _L, adj_vec, 0)
        lax.fori_loop(0, _NCHUNK, adj_chunk, 0)

        def zrow(b, _):
            def zcol(o, _):
                zbuf[b, pl.ds(o * _L, _L)] = zf
                return 0
            return lax.fori_loop(0, R, zcol, 0)
        lax.fori_loop(0, _ZR, zrow, 0)

        iota_div8 = lax.shift_right_arithmetic(iota, 3)
        iota_mod8 = lax.bitwise_and(iota, 7)
        iota_chan = lax.bitwise_and(iota_mod8, Hh - 1)

        def zd(g, _):
            r = jnp.full((_L,), g * 2, jnp.int32) + iota_div8
            plsc.store_scatter(zdbuf, [r, iota_mod8], zf)
            return 0
        lax.fori_loop(0, (srpt * _HP) // _L, zd, 0)

        for st in range(n_stripes):
            base = st * stride
            basev = jnp.full((_L,), base, jnp.int32)

            def zs(q, _):
                pltpu.sync_copy(zbuf, S.at[pl.ds(s * srpt + q * _ZR, _ZR)])
                return 0
            lax.fori_loop(0, srpt // _ZR, zs, 0)
            pltpu.sync_copy(zdbuf, D.at[pl.ds(s * srpt, srpt)])

            plsc.subcore_barrier()

            def chunk(j, _):
                sidx = src_adj.at[j]
                didx_a = dst_adj.at[j]
                cp_h = pltpu.async_copy(h_hbm.at[sidx], hbuf, sem_h)
                cp_es = pltpu.async_copy(es_hbm.at[sidx], esbuf, sem_e)
                cp_ed = pltpu.async_copy(ed_hbm.at[didx_a], edbuf, sem_d)

                def srow(q, _):
                    sl = pl.ds(q * _L, _L)
                    rel = dst_raw[j, sl] - basev
                    if n_stripes == 1:
                        dstrip[sl] = rel
                    else:
                        ok = jnp.logical_and(rel >= 0, rel < stride)
                        junk = jnp.full((_L,), stride, jnp.int32) + \
                            lax.bitwise_and(rel, 255)
                        dstrip[sl] = jnp.where(ok, rel, junk)
                    return 0
                lax.fori_loop(0, _B // _L, srow, 0)

                cp_es.wait()
                cp_ed.wait()

                def pstep(g, _):
                    r = jnp.full((_L,), g * 2, jnp.int32) + iota_div8
                    es_v = plsc.load_gather(esbuf, [r, iota_chan])
                    ed_v = plsc.load_gather(edbuf, [r, iota_chan])
                    e = es_v + ed_v
                    e = jnp.where(e >= 0.0, e, 0.2 * e)
                    plsc.store_scatter(pbuf, [r, iota_mod8],
                                       jnp.exp(e - m_vec))
                    return 0
                lax.fori_loop(0, (_B * _HP) // _L, pstep, 0)

                cp_h.wait()

                def mrow(b, _):
                    bv = jnp.full((_L,), b, jnp.int32)
                    def mcol(o, _):
                        off = o * _L
                        sl = pl.ds(off, _L)
                        head = lax.shift_right_arithmetic(
                            jnp.full((_L,), off, jnp.int32) + iota, log2C)
                        pv = plsc.load_gather(pbuf, [bv, head])
                        hbuf[b, sl] = hbuf[b, sl] * pv
                        return 0
                    return lax.fori_loop(0, R, mcol, 0)
                lax.fori_loop(0, _B, mrow, 0)

                pltpu.sync_copy(pbuf, D.at[dstrip], add=True)
                pltpu.sync_copy(hbuf, S.at[dstrip], add=True)
                return 0
            lax.fori_loop(0, _NCHUNK, chunk, 0)

            plsc.subcore_barrier()

            def rd(q, _):
                r0 = s * rpt + q * _ZR
                pltpu.sync_copy(S.at[pl.ds(r0, _ZR)], sbuf)
                pltpu.sync_copy(D.at[pl.ds(r0, _ZR)], dbuf)

                def rrow(b, _):
                    bv = jnp.full((_L,), b, jnp.int32)
                    def rcol(o, _):
                        off = o * _L
                        sl = pl.ds(off, _L)
                        head = lax.shift_right_arithmetic(
                            jnp.full((_L,), off, jnp.int32) + iota, log2C)
                        dv = plsc.load_gather(dbuf, [bv, head])
                        obuf[b, sl] = sbuf[b, sl] / (dv + 1e-16)
                        return 0
                    return lax.fori_loop(0, R, rcol, 0)
                lax.fori_loop(0, _ZR, rrow, 0)

                pltpu.sync_copy(obuf, out_hbm.at[c].at[pl.ds(base + r0, _ZR)])
                return 0
            lax.fori_loop(0, rpt // _ZR, rd, 0)

            if st + 1 < n_stripes:
                plsc.subcore_barrier()

    return k


_edge_sc1 = _make_edge_sc(_H1 // 2, _C1, (_H1 * _C1) // 2, 1)
_edge_sc2 = _make_edge_sc(1, _OUT, _OUT // 2, 2)


def _split_cols(t, Fh):
    n = t.shape[0]
    return t.reshape(n, 2, Fh).transpose(1, 0, 2).reshape(2 * n, Fh)


def _pad_heads(t):
    return jnp.pad(t, ((0, 0), (0, _HP - t.shape[1])))


def _shift_upper_bound(es, ed):
    m = jnp.max(es) + jnp.max(ed)
    m = jnp.where(m >= 0.0, m, 0.2 * m)
    return jnp.full((16,), m, jnp.float32)


def kernel(x, edge_index, W1, att_src1, att_dst1, b1, W2, att_src2, att_dst2, b2):
    loop = jnp.arange(_N, dtype=jnp.int32)
    pad = _ETP - (_E + _N)
    src = jnp.concatenate(
        [edge_index[0].astype(jnp.int32), loop, jnp.zeros((pad,), jnp.int32)])
    dst = jnp.concatenate(
        [edge_index[1].astype(jnp.int32), loop,
         jnp.full((pad,), _N, jnp.int32)])
    src2d = src.reshape(_NS, _NCHUNK, _B)
    dst2d = dst.reshape(_NS, _NCHUNK, _B)

    h1, es1, ed1 = _dense_stage(x, W1, att_src1, att_dst1, _H1, _C1)
    m1 = _shift_upper_bound(es1, ed1)
    h1s = _split_cols(h1, (_H1 * _C1) // 2)
    es1s = _pad_heads(_split_cols(es1, _H1 // 2))
    ed1p = jnp.concatenate([ed1, jnp.zeros((_NED - _N, _H1), jnp.float32)], 0)
    ed1s = _pad_heads(_split_cols(ed1p, _H1 // 2))
    agg1 = _edge_sc1(src2d, dst2d, h1s, es1s, ed1s, m1)
    agg1 = jnp.concatenate([agg1[0, :_N], agg1[1, :_N]], axis=1)

    h2, es2, ed2 = _dense_stage(agg1, W2, att_src2, att_dst2, 1, _OUT,
                                bias=b1)
    m2 = _shift_upper_bound(es2, ed2)
    h2s = _split_cols(h2, _OUT // 2)
    es2s = _pad_heads(jnp.concatenate([es2, es2], 0))
    ed2p = jnp.concatenate([ed2, jnp.zeros((_NED - _N, 1), jnp.float32)], 0)
    ed2s = _pad_heads(jnp.concatenate([ed2p, ed2p], 0))
    agg2 = _edge_sc2(src2d, dst2d, h2s, es2s, ed2s, m2)
    agg2 = jnp.concatenate([agg2[0, :_N], agg2[1, :_N]], axis=1)

    return _log_softmax_bias(agg2, b2)


# speedup vs baseline: 26.1383x; 1.3100x over previous
import functools

import jax
import jax.numpy as jnp
from jax import lax
from jax.experimental import pallas as pl
from jax.experimental.pallas import tpu as pltpu
from jax.experimental.pallas import tpu_sc as plsc

_N = 10000
_E = 320000
_IN = 128
_OUT = 128
_H1 = 8
_C1 = 8

_ROWS = 1000

_NC, _NS, _L = 2, 16, 16
_B = 128
_NCHUNK = 162
_ETP = _NS * _NCHUNK * _B
_NP = 10240
_RPT = _NP // _NS
_ZR = 32
_NED = _N + 8
_HP = 8


def _dense1_body(x_ref, w_ref, asrc_ref, adst_ref, h_ref, es_ref, ed_ref):
    h = jnp.dot(x_ref[...], w_ref[...], preferred_element_type=jnp.float32)
    h_ref[...] = h
    es_ref[...] = jnp.dot(h, asrc_ref[...], preferred_element_type=jnp.float32)
    ed_ref[...] = jnp.dot(h, adst_ref[...], preferred_element_type=jnp.float32)


def _dense2_body(x_ref, b_ref, w_ref, asrc_ref, adst_ref, h_ref, es_ref, ed_ref):
    o = jnp.maximum(x_ref[...] + b_ref[...], 0.0)
    h = jnp.dot(o, w_ref[...], preferred_element_type=jnp.float32)
    h_ref[...] = h
    es_ref[...] = jnp.dot(h, asrc_ref[...], preferred_element_type=jnp.float32)
    ed_ref[...] = jnp.dot(h, adst_ref[...], preferred_element_type=jnp.float32)


def _att_proj(a, heads, ch):
    eye = jnp.eye(heads, dtype=jnp.float32)
    return (a.reshape(heads, ch)[:, :, None] * eye[:, None, :]).reshape(
        heads * ch, heads)


def _dense_stage(x, W, a_src, a_dst, heads, ch, bias=None):
    n, k = x.shape[0], W.shape[1]
    Asrc = _att_proj(a_src, heads, ch)
    Adst = _att_proj(a_dst, heads, ch)
    grid = n // _ROWS
    in_specs = [
        pl.BlockSpec((_ROWS, x.shape[1]), lambda i: (i, 0)),
        pl.BlockSpec((x.shape[1], k), lambda i: (0, 0)),
        pl.BlockSpec((k, heads), lambda i: (0, 0)),
        pl.BlockSpec((k, heads), lambda i: (0, 0)),
    ]
    args = (x, W, Asrc, Adst)
    body = _dense1_body
    if bias is not None:
        in_specs.insert(1, pl.BlockSpec((1, x.shape[1]), lambda i: (0, 0)))
        args = (x, bias.reshape(1, -1), W, Asrc, Adst)
        body = _dense2_body
    return pl.pallas_call(
        body,
        grid=(grid,),
        in_specs=in_specs,
        out_specs=[
            pl.BlockSpec((_ROWS, k), lambda i: (i, 0)),
            pl.BlockSpec((_ROWS, heads), lambda i: (i, 0)),
            pl.BlockSpec((_ROWS, heads), lambda i: (i, 0)),
        ],
        out_shape=[
            jax.ShapeDtypeStruct((n, k), jnp.float32),
            jax.ShapeDtypeStruct((n, heads), jnp.float32),
            jax.ShapeDtypeStruct((n, heads), jnp.float32),
        ],
    )(*args)


def _lsm_body(x_ref, b_ref, o_ref):
    z = x_ref[...] + b_ref[...]
    m = jnp.max(z, axis=1, keepdims=True)
    ez = jnp.exp(z - m)
    lse = jnp.log(jnp.sum(ez, axis=1, keepdims=True))
    o_ref[...] = z - m - lse


def _log_softmax_bias(x, b):
    n, k = x.shape
    return pl.pallas_call(
        _lsm_body,
        grid=(n // _ROWS,),
        in_specs=[
            pl.BlockSpec((_ROWS, k), lambda i: (i, 0)),
            pl.BlockSpec((1, k), lambda i: (0, 0)),
        ],
        out_specs=pl.BlockSpec((_ROWS, k), lambda i: (i, 0)),
        out_shape=jax.ShapeDtypeStruct((n, k), jnp.float32),
    )(x, b.reshape(1, k))


def _make_edge_sc(Hh, C, Fh, n_stripes):
    R = Fh // _L
    log2C = C.bit_length() - 1
    stride = _NP // n_stripes
    srows = _NP if n_stripes == 1 else 5632
    srpt = srows // _NS
    rpt = stride // _NS
    junk0 = stride if n_stripes > 1 else _N + 8
    mesh = plsc.VectorSubcoreMesh(core_axis_name="c", subcore_axis_name="s")

    @functools.partial(
        pl.kernel,
        out_type=jax.ShapeDtypeStruct((2, _NP, Fh), jnp.float32),
        mesh=mesh,
        compiler_params=pltpu.CompilerParams(
            needs_layout_passes=False, use_tc_tiling_on_sc=False),
        scratch_types=[
            pltpu.VMEM((_NCHUNK, _B), jnp.int32),
            pltpu.VMEM((_NCHUNK, _B), jnp.int32),
            pltpu.VMEM((_NCHUNK, _B), jnp.int32),
            pltpu.VMEM((_B,), jnp.int32),
            pltpu.VMEM((_B,), jnp.int32),
            pltpu.VMEM((_B,), jnp.int32),
            pltpu.VMEM((_B, Fh), jnp.float32),
            pltpu.VMEM((_B, Fh), jnp.float32),
            pltpu.VMEM((_B, Fh), jnp.float32),
            pltpu.VMEM((_B, _HP), jnp.float32),
            pltpu.VMEM((_B, _HP), jnp.float32),
            pltpu.VMEM((_B, _HP), jnp.float32),
            pltpu.VMEM((_B, _HP), jnp.float32),
            pltpu.VMEM((_B, _HP), jnp.float32),
            pltpu.VMEM((_B, _HP), jnp.float32),
            pltpu.VMEM((_B, _HP), jnp.float32),
            pltpu.VMEM((_B, _HP), jnp.float32),
            pltpu.VMEM((_B, _HP), jnp.float32),
            pltpu.VMEM((_ZR, Fh), jnp.float32),
            pltpu.VMEM((srpt, _HP), jnp.float32),
            pltpu.VMEM((_ZR, Fh), jnp.float32),
            pltpu.VMEM((_ZR, _HP), jnp.float32),
            pltpu.VMEM((_ZR, Fh), jnp.float32),
            pltpu.VMEM((16,), jnp.float32),
            pltpu.VMEM_SHARED((srows, Fh), jnp.float32),
            pltpu.VMEM_SHARED((srows, _HP), jnp.float32),
            pltpu.SemaphoreType.DMA,
            pltpu.SemaphoreType.DMA,
            pltpu.SemaphoreType.DMA,
            pltpu.SemaphoreType.DMA,
            pltpu.SemaphoreType.DMA,
        ],
    )
    def k(src_hbm, dst_hbm, h_hbm, es_hbm, ed_hbm, m_hbm, out_hbm,
          src_adj, dst_raw, dst_adj,
          ds0, ds1, ds2, hb0, hb1, hb2, eA0, eA1, eA2, eB0, eB1, eB2,
          pb0, pb1, pb2,
          zbuf, zdbuf, sbuf, dbuf, obuf, mv, S, D,
          sem_h, sem_e, sem_d, sem_s, sem_p):
        c = lax.axis_index("c")
        s = lax.axis_index("s")
        iota = lax.iota(jnp.int32, _L)
        zf = jnp.zeros((_L,), jnp.float32)
        bufs = ((hb0, eA0, eB0, pb0, ds0),
                (hb1, eA1, eB1, pb1, ds1),
                (hb2, eA2, eB2, pb2, ds2))

        pltpu.sync_copy(src_hbm.at[s], src_adj)
        pltpu.sync_copy(dst_hbm.at[s], dst_raw)
        pltpu.sync_copy(m_hbm, mv)
        m_vec = mv[...]

        cNv = jnp.full((_L,), c * _N, jnp.int32)
        cMv = jnp.full((_L,), c * _NED, jnp.int32)

        def adj_chunk(j, _):
            def adj_vec(q, _):
                sl = pl.ds(q * _L, _L)
                src_adj[j, sl] = src_adj[j, sl] + cNv
                dst_adj[j, sl] = dst_raw[j, sl] + cMv
                return 0
            return lax.fori_loop(0, _B // _L, adj_vec, 0)
        lax.fori_loop(0, _NCHUNK, adj_chunk, 0)

        def zrow(b, _):
            def zcol(o, _):
                zbuf[b, pl.ds(o * _L, _L)] = zf
                return 0
            return lax.fori_loop(0, R, zcol, 0)
        lax.fori_loop(0, _ZR, zrow, 0)

        iota_div8 = lax.shift_right_arithmetic(iota, 3)
        iota_mod8 = lax.bitwise_and(iota, 7)
        iota_chan = lax.bitwise_and(iota_mod8, Hh - 1)

        def zd(g, _):
            r = jnp.full((_L,), g * 2, jnp.int32) + iota_div8
            plsc.store_scatter(zdbuf, [r, iota_mod8], zf)
            return 0
        lax.fori_loop(0, (srpt * _HP) // _L, zd, 0)

        junkv = jnp.full((_L,), junk0, jnp.int32)

        def gather_issue(j, b):
            hb, eA, eB, pbn, dsb = bufs[b]
            sidx = src_adj.at[j]
            pltpu.async_copy(h_hbm.at[sidx], hb, sem_h)
            pltpu.async_copy(es_hbm.at[sidx], eA, sem_e)
            pltpu.async_copy(ed_hbm.at[dst_adj.at[j]], eB, sem_d)

        def scatter_issue(b):
            hb, eA, eB, pbn, dsb = bufs[b]
            pltpu.async_copy(hb, S.at[dsb], sem_s, add=True)
            pltpu.async_copy(pbn, D.at[dsb], sem_p, add=True)

        def scatter_drain(b):
            hb, eA, eB, pbn, dsb = bufs[b]
            pltpu.make_async_copy(hb, S.at[dsb], sem_s).wait()
            pltpu.make_async_copy(pbn, D.at[dsb], sem_p).wait()

        for st in range(n_stripes):
            base = st * stride
            basev = jnp.full((_L,), base, jnp.int32)

            def zs(q, _):
                pltpu.sync_copy(zbuf, S.at[pl.ds(s * srpt + q * _ZR, _ZR)])
                return 0
            lax.fori_loop(0, srpt // _ZR, zs, 0)
            pltpu.sync_copy(zdbuf, D.at[pl.ds(s * srpt, srpt)])

            plsc.subcore_barrier()

            def substep(j, b):
                hb, eA, eB, pbn, dsb = bufs[b]
                nb = (b + 1) % 3
                scatter_drain(nb)
                gather_issue(jnp.minimum(j + 1, _NCHUNK - 1), nb)

                def srow(q, _):
                    sl = pl.ds(q * _L, _L)
                    rel = dst_raw[j, sl] - basev
                    if n_stripes == 1:
                        dsb[sl] = rel
                    else:
                        ok = jnp.logical_and(rel >= 0, rel < stride)
                        junk = junkv + lax.bitwise_and(rel, 255)
                        dsb[sl] = jnp.where(ok, rel, junk)
                    return 0
                lax.fori_loop(0, _B // _L, srow, 0)

                sidx = src_adj.at[j]
                pltpu.make_async_copy(es_hbm.at[sidx], eA, sem_e).wait()
                pltpu.make_async_copy(ed_hbm.at[dst_adj.at[j]], eB, sem_d).wait()

                def pstep(g, _):
                    r = jnp.full((_L,), g * 2, jnp.int32) + iota_div8
                    es_v = plsc.load_gather(eA, [r, iota_chan])
                    ed_v = plsc.load_gather(eB, [r, iota_chan])
                    e = es_v + ed_v
                    e = jnp.where(e >= 0.0, e, 0.2 * e)
                    plsc.store_scatter(pbn, [r, iota_mod8],
                                       jnp.exp(e - m_vec))
                    return 0
                lax.fori_loop(0, (_B * _HP) // _L, pstep, 0)

                pltpu.make_async_copy(h_hbm.at[sidx], hb, sem_h).wait()

                def mrow(bb, _):
                    bv = jnp.full((_L,), bb, jnp.int32)
                    def mcol(o, _):
                        off = o * _L
                        sl = pl.ds(off, _L)
                        head = lax.shift_right_arithmetic(
                            jnp.full((_L,), off, jnp.int32) + iota, log2C)
                        pv = plsc.load_gather(pbn, [bv, head])
                        hb[bb, sl] = hb[bb, sl] * pv
                        return 0
                    return lax.fori_loop(0, R, mcol, 0)
                lax.fori_loop(0, _B, mrow, 0)

                scatter_issue(b)

            def dfill(b):
                dsb = bufs[b][4]
                def f(q, _):
                    dsb[pl.ds(q * _L, _L)] = junkv
                    return 0
                lax.fori_loop(0, _B // _L, f, 0)
            dfill(1)
            dfill(2)
            scatter_issue(1)
            scatter_issue(2)
            gather_issue(jnp.int32(0), 0)

            def tri(g, _):
                j = g * 3
                substep(j, 0)
                substep(j + 1, 1)
                substep(j + 2, 2)
                return 0
            lax.fori_loop(0, _NCHUNK // 3, tri, 0)

            scatter_drain(1)
            scatter_drain(2)
            hb, eA, eB, pbn, dsb = bufs[0]
            jl = jnp.int32(_NCHUNK - 1)
            pltpu.make_async_copy(es_hbm.at[src_adj.at[jl]], eA, sem_e).wait()
            pltpu.make_async_copy(ed_hbm.at[dst_adj.at[jl]], eB, sem_d).wait()
            pltpu.make_async_copy(h_hbm.at[src_adj.at[jl]], hb, sem_h).wait()

            plsc.subcore_barrier()

            def rd(q, _):
                r0 = s * rpt + q * _ZR
                pltpu.sync_copy(S.at[pl.ds(r0, _ZR)], sbuf)
                pltpu.sync_copy(D.at[pl.ds(r0, _ZR)], dbuf)

                def rrow(b, _):
                    bv = jnp.full((_L,), b, jnp.int32)
                    def rcol(o, _):
                        off = o * _L
                        sl = pl.ds(off, _L)
                        head = lax.shift_right_arithmetic(
                            jnp.full((_L,), off, jnp.int32) + iota, log2C)
                        dv = plsc.load_gather(dbuf, [bv, head])
                        obuf[b, sl] = sbuf[b, sl] / (dv + 1e-16)
                        return 0
                    return lax.fori_loop(0, R, rcol, 0)
                lax.fori_loop(0, _ZR, rrow, 0)

                pltpu.sync_copy(obuf, out_hbm.at[c].at[pl.ds(base + r0, _ZR)])
                return 0
            lax.fori_loop(0, rpt // _ZR, rd, 0)

            if st + 1 < n_stripes:
                plsc.subcore_barrier()

    return k


_edge_sc1 = _make_edge_sc(_H1 // 2, _C1, (_H1 * _C1) // 2, 1)
_edge_sc2 = _make_edge_sc(1, _OUT, _OUT // 2, 2)


def _split_cols(t, Fh):
    n = t.shape[0]
    return t.reshape(n, 2, Fh).transpose(1, 0, 2).reshape(2 * n, Fh)


def _pad_heads(t):
    return jnp.pad(t, ((0, 0), (0, _HP - t.shape[1])))


def _shift_upper_bound(es, ed):
    m = jnp.max(es) + jnp.max(ed)
    m = jnp.where(m >= 0.0, m, 0.2 * m)
    return jnp.full((16,), m, jnp.float32)


def kernel(x, edge_index, W1, att_src1, att_dst1, b1, W2, att_src2, att_dst2, b2):
    loop = jnp.arange(_N, dtype=jnp.int32)
    pad = _ETP - (_E + _N)
    src = jnp.concatenate(
        [edge_index[0].astype(jnp.int32), loop, jnp.zeros((pad,), jnp.int32)])
    dst = jnp.concatenate(
        [edge_index[1].astype(jnp.int32), loop,
         jnp.full((pad,), _N, jnp.int32)])
    src2d = src.reshape(_NS, _NCHUNK, _B)
    dst2d = dst.reshape(_NS, _NCHUNK, _B)

    h1, es1, ed1 = _dense_stage(x, W1, att_src1, att_dst1, _H1, _C1)
    m1 = _shift_upper_bound(es1, ed1)
    h1s = _split_cols(h1, (_H1 * _C1) // 2)
    es1s = _pad_heads(_split_cols(es1, _H1 // 2))
    ed1p = jnp.concatenate([ed1, jnp.zeros((_NED - _N, _H1), jnp.float32)], 0)
    ed1s = _pad_heads(_split_cols(ed1p, _H1 // 2))
    agg1 = _edge_sc1(src2d, dst2d, h1s, es1s, ed1s, m1)
    agg1 = jnp.concatenate([agg1[0, :_N], agg1[1, :_N]], axis=1)

    h2, es2, ed2 = _dense_stage(agg1, W2, att_src2, att_dst2, 1, _OUT,
                                bias=b1)
    m2 = _shift_upper_bound(es2, ed2)
    h2s = _split_cols(h2, _OUT // 2)
    es2s = _pad_heads(jnp.concatenate([es2, es2], 0))
    ed2p = jnp.concatenate([ed2, jnp.zeros((_NED - _N, 1), jnp.float32)], 0)
    ed2s = _pad_heads(jnp.concatenate([ed2p, ed2p], 0))
    agg2 = _edge_sc2(src2d, dst2d, h2s, es2s, ed2s, m2)
    agg2 = jnp.concatenate([agg2[0, :_N], agg2[1, :_N]], axis=1)

    return _log_softmax_bias(agg2, b2)


# speedup vs baseline: 35.1644x; 1.3453x over previous
import functools

import jax
import jax.numpy as jnp
from jax import lax
from jax.experimental import pallas as pl
from jax.experimental.pallas import tpu as pltpu
from jax.experimental.pallas import tpu_sc as plsc

_N = 10000
_E = 320000
_IN = 128
_OUT = 128
_H1 = 8
_C1 = 8

_ROWS = 1000

_NC, _NS, _L = 2, 16, 16
_B = 128
_NCHUNK = 162
_ETP = _NS * _NCHUNK * _B
_NP = 10240
_RPT = _NP // _NS
_ZR = 32
_NED = _N + 8
_HP = 8


def _dense1_body(x_ref, w_ref, asrc_ref, adst_ref, h_ref, es_ref, ed_ref):
    h = jnp.dot(x_ref[...], w_ref[...], preferred_element_type=jnp.float32)
    h_ref[...] = h
    es_ref[...] = jnp.dot(h, asrc_ref[...], preferred_element_type=jnp.float32)
    ed_ref[...] = jnp.dot(h, adst_ref[...], preferred_element_type=jnp.float32)


def _dense2_body(x_ref, b_ref, w_ref, asrc_ref, adst_ref, h_ref, es_ref, ed_ref):
    o = jnp.maximum(x_ref[...] + b_ref[...], 0.0)
    h = jnp.dot(o, w_ref[...], preferred_element_type=jnp.float32)
    h_ref[...] = h
    es_ref[...] = jnp.dot(h, asrc_ref[...], preferred_element_type=jnp.float32)
    ed_ref[...] = jnp.dot(h, adst_ref[...], preferred_element_type=jnp.float32)


def _att_proj(a, heads, ch):
    eye = jnp.eye(heads, dtype=jnp.float32)
    return (a.reshape(heads, ch)[:, :, None] * eye[:, None, :]).reshape(
        heads * ch, heads)


def _dense_stage(x, W, a_src, a_dst, heads, ch, bias=None):
    n, k = x.shape[0], W.shape[1]
    Asrc = _att_proj(a_src, heads, ch)
    Adst = _att_proj(a_dst, heads, ch)
    grid = n // _ROWS
    in_specs = [
        pl.BlockSpec((_ROWS, x.shape[1]), lambda i: (i, 0)),
        pl.BlockSpec((x.shape[1], k), lambda i: (0, 0)),
        pl.BlockSpec((k, heads), lambda i: (0, 0)),
        pl.BlockSpec((k, heads), lambda i: (0, 0)),
    ]
    args = (x, W, Asrc, Adst)
    body = _dense1_body
    if bias is not None:
        in_specs.insert(1, pl.BlockSpec((1, x.shape[1]), lambda i: (0, 0)))
        args = (x, bias.reshape(1, -1), W, Asrc, Adst)
        body = _dense2_body
    return pl.pallas_call(
        body,
        grid=(grid,),
        in_specs=in_specs,
        out_specs=[
            pl.BlockSpec((_ROWS, k), lambda i: (i, 0)),
            pl.BlockSpec((_ROWS, heads), lambda i: (i, 0)),
            pl.BlockSpec((_ROWS, heads), lambda i: (i, 0)),
        ],
        out_shape=[
            jax.ShapeDtypeStruct((n, k), jnp.float32),
            jax.ShapeDtypeStruct((n, heads), jnp.float32),
            jax.ShapeDtypeStruct((n, heads), jnp.float32),
        ],
    )(*args)


def _lsm_body(x_ref, b_ref, o_ref):
    z = x_ref[...] + b_ref[...]
    m = jnp.max(z, axis=1, keepdims=True)
    ez = jnp.exp(z - m)
    lse = jnp.log(jnp.sum(ez, axis=1, keepdims=True))
    o_ref[...] = z - m - lse


def _log_softmax_bias(x, b):
    n, k = x.shape
    return pl.pallas_call(
        _lsm_body,
        grid=(n // _ROWS,),
        in_specs=[
            pl.BlockSpec((_ROWS, k), lambda i: (i, 0)),
            pl.BlockSpec((1, k), lambda i: (0, 0)),
        ],
        out_specs=pl.BlockSpec((_ROWS, k), lambda i: (i, 0)),
        out_shape=jax.ShapeDtypeStruct((n, k), jnp.float32),
    )(x, b.reshape(1, k))


def _make_edge_sc(Hh, C, Fh, n_stripes):
    R = Fh // _L
    log2C = C.bit_length() - 1
    stride = _NP // n_stripes
    srows = _NP if n_stripes == 1 else 5632
    srpt = srows // _NS
    rpt = stride // _NS
    junk0 = stride if n_stripes > 1 else _N + 8
    mesh = plsc.VectorSubcoreMesh(core_axis_name="c", subcore_axis_name="s")

    @functools.partial(
        pl.kernel,
        out_type=jax.ShapeDtypeStruct((2, _NP, Fh), jnp.float32),
        mesh=mesh,
        compiler_params=pltpu.CompilerParams(
            needs_layout_passes=False, use_tc_tiling_on_sc=False),
        scratch_types=[
            pltpu.VMEM((_NCHUNK, _B), jnp.int32),
            pltpu.VMEM((_NCHUNK, _B), jnp.int32),
            pltpu.VMEM((_NCHUNK, _B), jnp.int32),
            pltpu.VMEM((_B,), jnp.int32),
            pltpu.VMEM((_B,), jnp.int32),
            pltpu.VMEM((_B,), jnp.int32),
            pltpu.VMEM((_B, Fh), jnp.float32),
            pltpu.VMEM((_B, Fh), jnp.float32),
            pltpu.VMEM((_B, Fh), jnp.float32),
            pltpu.VMEM((_B, _HP), jnp.float32),
            pltpu.VMEM((_B, _HP), jnp.float32),
            pltpu.VMEM((_B, _HP), jnp.float32),
            pltpu.VMEM((_B, _HP), jnp.float32),
            pltpu.VMEM((_B, _HP), jnp.float32),
            pltpu.VMEM((_B, _HP), jnp.float32),
            pltpu.VMEM((_B, _HP), jnp.float32),
            pltpu.VMEM((_B, _HP), jnp.float32),
            pltpu.VMEM((_B, _HP), jnp.float32),
            pltpu.VMEM((_ZR, Fh), jnp.float32),
            pltpu.VMEM((srpt, _HP), jnp.float32),
            pltpu.VMEM((_ZR, Fh), jnp.float32),
            pltpu.VMEM((_ZR, _HP), jnp.float32),
            pltpu.VMEM((_ZR, Fh), jnp.float32),
            pltpu.VMEM((16,), jnp.float32),
            pltpu.VMEM_SHARED((srows, Fh), jnp.float32),
            pltpu.VMEM_SHARED((srows, _HP), jnp.float32),
            pltpu.SemaphoreType.DMA,
            pltpu.SemaphoreType.DMA,
            pltpu.SemaphoreType.DMA,
            pltpu.SemaphoreType.DMA,
            pltpu.SemaphoreType.DMA,
        ],
    )
    def k(src_hbm, dst_hbm, h_hbm, es_hbm, ed_hbm, m_hbm, out_hbm,
          src_adj, dst_raw, dst_adj,
          ds0, ds1, ds2, hb0, hb1, hb2, eA0, eA1, eA2, eB0, eB1, eB2,
          pb0, pb1, pb2,
          zbuf, zdbuf, sbuf, dbuf, obuf, mv, S, D,
          sem_h, sem_e, sem_d, sem_s, sem_p):
        c = lax.axis_index("c")
        s = lax.axis_index("s")
        iota = lax.iota(jnp.int32, _L)
        zf = jnp.zeros((_L,), jnp.float32)
        bufs = ((hb0, eA0, eB0, pb0, ds0),
                (hb1, eA1, eB1, pb1, ds1),
                (hb2, eA2, eB2, pb2, ds2))

        pltpu.sync_copy(src_hbm.at[s], src_adj)
        pltpu.sync_copy(dst_hbm.at[s], dst_raw)
        pltpu.sync_copy(m_hbm, mv)
        m_vec = mv[...]

        cNv = jnp.full((_L,), c * _N, jnp.int32)
        cMv = jnp.full((_L,), c * _NED, jnp.int32)

        def adj_chunk(j, _):
            def adj_vec(q, _):
                sl = pl.ds(q * _L, _L)
                src_adj[j, sl] = src_adj[j, sl] + cNv
                dst_adj[j, sl] = dst_raw[j, sl] + cMv
                return 0
            return lax.fori_loop(0, _B // _L, adj_vec, 0)
        lax.fori_loop(0, _NCHUNK, adj_chunk, 0)

        def zrow(b, _):
            def zcol(o, _):
                zbuf[b, pl.ds(o * _L, _L)] = zf
                return 0
            return lax.fori_loop(0, R, zcol, 0)
        lax.fori_loop(0, _ZR, zrow, 0)

        iota_div8 = lax.shift_right_arithmetic(iota, 3)
        iota_mod8 = lax.bitwise_and(iota, 7)
        iota_chan = lax.bitwise_and(iota_mod8, Hh - 1)

        def zd(g, _):
            r = jnp.full((_L,), g * 2, jnp.int32) + iota_div8
            plsc.store_scatter(zdbuf, [r, iota_mod8], zf)
            return 0
        lax.fori_loop(0, (srpt * _HP) // _L, zd, 0)

        junkv = jnp.full((_L,), junk0, jnp.int32)

        def gather_issue(j, b):
            hb, eA, eB, pbn, dsb = bufs[b]
            sidx = src_adj.at[j]
            pltpu.async_copy(h_hbm.at[sidx], hb, sem_h)
            pltpu.async_copy(es_hbm.at[sidx], eA, sem_e)
            pltpu.async_copy(ed_hbm.at[dst_adj.at[j]], eB, sem_d)

        def scatter_issue(b):
            hb, eA, eB, pbn, dsb = bufs[b]
            pltpu.async_copy(hb, S.at[dsb], sem_s, add=True)
            pltpu.async_copy(pbn, D.at[dsb], sem_p, add=True)

        def scatter_drain(b):
            hb, eA, eB, pbn, dsb = bufs[b]
            pltpu.make_async_copy(hb, S.at[dsb], sem_s).wait()
            pltpu.make_async_copy(pbn, D.at[dsb], sem_p).wait()

        for st in range(n_stripes):
            base = st * stride
            basev = jnp.full((_L,), base, jnp.int32)

            def zs(q, _):
                pltpu.sync_copy(zbuf, S.at[pl.ds(s * srpt + q * _ZR, _ZR)])
                return 0
            lax.fori_loop(0, srpt // _ZR, zs, 0)
            pltpu.sync_copy(zdbuf, D.at[pl.ds(s * srpt, srpt)])

            plsc.subcore_barrier()

            def substep(j, b):
                hb, eA, eB, pbn, dsb = bufs[b]
                nb = (b + 1) % 3
                scatter_drain(nb)
                gather_issue(jnp.minimum(j + 1, _NCHUNK - 1), nb)

                def srow(q, _):
                    sl = pl.ds(q * _L, _L)
                    rel = dst_raw[j, sl] - basev
                    if n_stripes == 1:
                        dsb[sl] = rel
                    else:
                        ok = jnp.logical_and(rel >= 0, rel < stride)
                        junk = junkv + lax.bitwise_and(rel, 255)
                        dsb[sl] = jnp.where(ok, rel, junk)
                    return 0
                lax.fori_loop(0, _B // _L, srow, 0)

                sidx = src_adj.at[j]
                pltpu.make_async_copy(es_hbm.at[sidx], eA, sem_e).wait()
                pltpu.make_async_copy(ed_hbm.at[dst_adj.at[j]], eB, sem_d).wait()

                def pstep(g, _):
                    r = jnp.full((_L,), g * 2, jnp.int32) + iota_div8
                    es_v = plsc.load_gather(eA, [r, iota_chan])
                    ed_v = plsc.load_gather(eB, [r, iota_chan])
                    e = es_v + ed_v
                    e = jnp.where(e >= 0.0, e, 0.2 * e)
                    plsc.store_scatter(pbn, [r, iota_mod8],
                                       jnp.exp(e - m_vec))
                    return 0
                lax.fori_loop(0, (_B * _HP) // _L, pstep, 0)

                pltpu.make_async_copy(h_hbm.at[sidx], hb, sem_h).wait()

                if Hh == 1:
                    zi = jnp.zeros((_L,), jnp.int32)
                    def mrow(bb, _):
                        bv = jnp.full((_L,), bb, jnp.int32)
                        pv = plsc.load_gather(pbn, [bv, zi])
                        def mcol(o, _):
                            sl = pl.ds(o * _L, _L)
                            hb[bb, sl] = hb[bb, sl] * pv
                            return 0
                        return lax.fori_loop(0, R, mcol, 0)
                else:
                    def mrow(bb, _):
                        bv = jnp.full((_L,), bb, jnp.int32)
                        def mcol(o, _):
                            off = o * _L
                            sl = pl.ds(off, _L)
                            head = lax.shift_right_arithmetic(
                                jnp.full((_L,), off, jnp.int32) + iota, log2C)
                            pv = plsc.load_gather(pbn, [bv, head])
                            hb[bb, sl] = hb[bb, sl] * pv
                            return 0
                        return lax.fori_loop(0, R, mcol, 0)
                lax.fori_loop(0, _B, mrow, 0)

                scatter_issue(b)

            def dfill(b):
                dsb = bufs[b][4]
                def f(q, _):
                    dsb[pl.ds(q * _L, _L)] = junkv
                    return 0
                lax.fori_loop(0, _B // _L, f, 0)
            dfill(1)
            dfill(2)
            scatter_issue(1)
            scatter_issue(2)
            gather_issue(jnp.int32(0), 0)

            def tri(g, _):
                j = g * 3
                substep(j, 0)
                substep(j + 1, 1)
                substep(j + 2, 2)
                return 0
            lax.fori_loop(0, _NCHUNK // 3, tri, 0)

            scatter_drain(1)
            scatter_drain(2)
            hb, eA, eB, pbn, dsb = bufs[0]
            jl = jnp.int32(_NCHUNK - 1)
            pltpu.make_async_copy(es_hbm.at[src_adj.at[jl]], eA, sem_e).wait()
            pltpu.make_async_copy(ed_hbm.at[dst_adj.at[jl]], eB, sem_d).wait()
            pltpu.make_async_copy(h_hbm.at[src_adj.at[jl]], hb, sem_h).wait()

            plsc.subcore_barrier()

            def rd(q, _):
                r0 = s * rpt + q * _ZR
                pltpu.sync_copy(S.at[pl.ds(r0, _ZR)], sbuf)
                pltpu.sync_copy(D.at[pl.ds(r0, _ZR)], dbuf)

                if Hh == 1:
                    zi = jnp.zeros((_L,), jnp.int32)
                    def rrow(b, _):
                        bv = jnp.full((_L,), b, jnp.int32)
                        dv = plsc.load_gather(dbuf, [bv, zi]) + 1e-16
                        def rcol(o, _):
                            sl = pl.ds(o * _L, _L)
                            obuf[b, sl] = sbuf[b, sl] / dv
                            return 0
                        return lax.fori_loop(0, R, rcol, 0)
                else:
                    def rrow(b, _):
                        bv = jnp.full((_L,), b, jnp.int32)
                        def rcol(o, _):
                            off = o * _L
                            sl = pl.ds(off, _L)
                            head = lax.shift_right_arithmetic(
                                jnp.full((_L,), off, jnp.int32) + iota, log2C)
                            dv = plsc.load_gather(dbuf, [bv, head])
                            obuf[b, sl] = sbuf[b, sl] / (dv + 1e-16)
                            return 0
                        return lax.fori_loop(0, R, rcol, 0)
                lax.fori_loop(0, _ZR, rrow, 0)

                pltpu.sync_copy(obuf, out_hbm.at[c].at[pl.ds(base + r0, _ZR)])
                return 0
            lax.fori_loop(0, rpt // _ZR, rd, 0)

            if st + 1 < n_stripes:
                plsc.subcore_barrier()

    return k


_edge_sc1 = _make_edge_sc(_H1 // 2, _C1, (_H1 * _C1) // 2, 1)
_edge_sc2 = _make_edge_sc(1, _OUT, _OUT // 2, 2)


def _split_cols(t, Fh):
    n = t.shape[0]
    return t.reshape(n, 2, Fh).transpose(1, 0, 2).reshape(2 * n, Fh)


def _pad_heads(t):
    return jnp.pad(t, ((0, 0), (0, _HP - t.shape[1])))


def _shift_upper_bound(es, ed):
    m = jnp.max(es) + jnp.max(ed)
    m = jnp.where(m >= 0.0, m, 0.2 * m)
    return jnp.full((16,), m, jnp.float32)


def kernel(x, edge_index, W1, att_src1, att_dst1, b1, W2, att_src2, att_dst2, b2):
    loop = jnp.arange(_N, dtype=jnp.int32)
    pad = _ETP - (_E + _N)
    src = jnp.concatenate(
        [edge_index[0].astype(jnp.int32), loop, jnp.zeros((pad,), jnp.int32)])
    dst = jnp.concatenate(
        [edge_index[1].astype(jnp.int32), loop,
         jnp.full((pad,), _N, jnp.int32)])
    src2d = src.reshape(_NS, _NCHUNK, _B)
    dst2d = dst.reshape(_NS, _NCHUNK, _B)

    h1, es1, ed1 = _dense_stage(x, W1, att_src1, att_dst1, _H1, _C1)
    m1 = _shift_upper_bound(es1, ed1)
    h1s = _split_cols(h1, (_H1 * _C1) // 2)
    es1s = _pad_heads(_split_cols(es1, _H1 // 2))
    ed1p = jnp.concatenate([ed1, jnp.zeros((_NED - _N, _H1), jnp.float32)], 0)
    ed1s = _pad_heads(_split_cols(ed1p, _H1 // 2))
    agg1 = _edge_sc1(src2d, dst2d, h1s, es1s, ed1s, m1)
    agg1 = jnp.concatenate([agg1[0, :_N], agg1[1, :_N]], axis=1)

    h2, es2, ed2 = _dense_stage(agg1, W2, att_src2, att_dst2, 1, _OUT,
                                bias=b1)
    m2 = _shift_upper_bound(es2, ed2)
    h2s = _split_cols(h2, _OUT // 2)
    es2s = _pad_heads(jnp.concatenate([es2, es2], 0))
    ed2p = jnp.concatenate([ed2, jnp.zeros((_NED - _N, 1), jnp.float32)], 0)
    ed2s = _pad_heads(jnp.concatenate([ed2p, ed2p], 0))
    agg2 = _edge_sc2(src2d, dst2d, h2s, es2s, ed2s, m2)
    agg2 = jnp.concatenate([agg2[0, :_N], agg2[1, :_N]], axis=1)

    return _log_softmax_bias(agg2, b2)


# speedup vs baseline: 44.9711x; 1.2789x over previous
import functools

import jax
import jax.numpy as jnp
from jax import lax
from jax.experimental import pallas as pl
from jax.experimental.pallas import tpu as pltpu
from jax.experimental.pallas import tpu_sc as plsc

_N = 10000
_E = 320000
_IN = 128
_OUT = 128
_H1 = 8
_C1 = 8

_ROWS = 1000

_NC, _NS, _L = 2, 16, 16
_B = 128
_NCHUNK = 162
_ETP = _NS * _NCHUNK * _B
_NP = 10240
_RPT = _NP // _NS
_ZR = 32
_NED = _N + 8
_HP = 8


def _dense1_body(x_ref, w_ref, asrc_ref, adst_ref, h_ref, es_ref, ed_ref):
    h = jnp.dot(x_ref[...], w_ref[...], preferred_element_type=jnp.float32)
    h_ref[...] = h
    es_ref[...] = jnp.dot(h, asrc_ref[...], preferred_element_type=jnp.float32)
    ed_ref[...] = jnp.dot(h, adst_ref[...], preferred_element_type=jnp.float32)


def _dense2_body(x_ref, b_ref, w_ref, asrc_ref, adst_ref, h_ref, es_ref, ed_ref):
    o = jnp.maximum(x_ref[...] + b_ref[...], 0.0)
    h = jnp.dot(o, w_ref[...], preferred_element_type=jnp.float32)
    h_ref[...] = h
    es_ref[...] = jnp.dot(h, asrc_ref[...], preferred_element_type=jnp.float32)
    ed_ref[...] = jnp.dot(h, adst_ref[...], preferred_element_type=jnp.float32)


def _att_proj(a, heads, ch):
    eye = jnp.eye(heads, dtype=jnp.float32)
    return (a.reshape(heads, ch)[:, :, None] * eye[:, None, :]).reshape(
        heads * ch, heads)


def _dense_stage(x, W, a_src, a_dst, heads, ch, bias=None):
    n, k = x.shape[0], W.shape[1]
    Asrc = _att_proj(a_src, heads, ch)
    Adst = _att_proj(a_dst, heads, ch)
    grid = n // _ROWS
    in_specs = [
        pl.BlockSpec((_ROWS, x.shape[1]), lambda i: (i, 0)),
        pl.BlockSpec((x.shape[1], k), lambda i: (0, 0)),
        pl.BlockSpec((k, heads), lambda i: (0, 0)),
        pl.BlockSpec((k, heads), lambda i: (0, 0)),
    ]
    args = (x, W, Asrc, Adst)
    body = _dense1_body
    if bias is not None:
        in_specs.insert(1, pl.BlockSpec((1, x.shape[1]), lambda i: (0, 0)))
        args = (x, bias.reshape(1, -1), W, Asrc, Adst)
        body = _dense2_body
    return pl.pallas_call(
        body,
        grid=(grid,),
        in_specs=in_specs,
        out_specs=[
            pl.BlockSpec((_ROWS, k), lambda i: (i, 0)),
            pl.BlockSpec((_ROWS, heads), lambda i: (i, 0)),
            pl.BlockSpec((_ROWS, heads), lambda i: (i, 0)),
        ],
        out_shape=[
            jax.ShapeDtypeStruct((n, k), jnp.float32),
            jax.ShapeDtypeStruct((n, heads), jnp.float32),
            jax.ShapeDtypeStruct((n, heads), jnp.float32),
        ],
    )(*args)


def _lsm_body(x_ref, b_ref, o_ref):
    z = x_ref[...] + b_ref[...]
    m = jnp.max(z, axis=1, keepdims=True)
    ez = jnp.exp(z - m)
    lse = jnp.log(jnp.sum(ez, axis=1, keepdims=True))
    o_ref[...] = z - m - lse


def _log_softmax_bias(x, b):
    n, k = x.shape
    return pl.pallas_call(
        _lsm_body,
        grid=(n // _ROWS,),
        in_specs=[
            pl.BlockSpec((_ROWS, k), lambda i: (i, 0)),
            pl.BlockSpec((1, k), lambda i: (0, 0)),
        ],
        out_specs=pl.BlockSpec((_ROWS, k), lambda i: (i, 0)),
        out_shape=jax.ShapeDtypeStruct((n, k), jnp.float32),
    )(x, b.reshape(1, k))


def _make_edge_sc(Hh, C, Fh, n_stripes):
    R = Fh // _L
    log2C = C.bit_length() - 1
    stride = _NP // n_stripes
    srows = _NP if n_stripes == 1 else 5632
    srpt = srows // _NS
    rpt = stride // _NS
    junk0 = stride if n_stripes > 1 else _N + 8
    mesh = plsc.VectorSubcoreMesh(core_axis_name="c", subcore_axis_name="s")

    @functools.partial(
        pl.kernel,
        out_type=jax.ShapeDtypeStruct((2, _NP, Fh), jnp.float32),
        mesh=mesh,
        compiler_params=pltpu.CompilerParams(
            needs_layout_passes=False, use_tc_tiling_on_sc=False),
        scratch_types=[
            pltpu.VMEM((_NCHUNK, _B), jnp.int32),
            pltpu.VMEM((_NCHUNK, _B), jnp.int32),
            pltpu.VMEM((_NCHUNK, _B), jnp.int32),
            pltpu.VMEM((_B,), jnp.int32),
            pltpu.VMEM((_B,), jnp.int32),
            pltpu.VMEM((_B,), jnp.int32),
            pltpu.VMEM((_B, Fh), jnp.float32),
            pltpu.VMEM((_B, Fh), jnp.float32),
            pltpu.VMEM((_B, Fh), jnp.float32),
            pltpu.VMEM((_B, _HP), jnp.float32),
            pltpu.VMEM((_B, _HP), jnp.float32),
            pltpu.VMEM((_B, _HP), jnp.float32),
            pltpu.VMEM((_B, _HP), jnp.float32),
            pltpu.VMEM((_B, _HP), jnp.float32),
            pltpu.VMEM((_B, _HP), jnp.float32),
            pltpu.VMEM((_B, _HP), jnp.float32),
            pltpu.VMEM((_B, _HP), jnp.float32),
            pltpu.VMEM((_B, _HP), jnp.float32),
            pltpu.VMEM((_ZR, Fh), jnp.float32),
            pltpu.VMEM((srpt, _HP), jnp.float32),
            pltpu.VMEM((_ZR, Fh), jnp.float32),
            pltpu.VMEM((_ZR, _HP), jnp.float32),
            pltpu.VMEM((_ZR, Fh), jnp.float32),
            pltpu.VMEM((16,), jnp.float32),
            pltpu.VMEM_SHARED((srows, Fh), jnp.float32),
            pltpu.VMEM_SHARED((srows, _HP), jnp.float32),
            pltpu.SemaphoreType.DMA,
            pltpu.SemaphoreType.DMA,
            pltpu.SemaphoreType.DMA,
            pltpu.SemaphoreType.DMA,
            pltpu.SemaphoreType.DMA,
        ],
    )
    def k(src_hbm, dst_hbm, h_hbm, es_hbm, ed_hbm, m_hbm, out_hbm,
          src_adj, dst_raw, dst_adj,
          ds0, ds1, ds2, hb0, hb1, hb2, eA0, eA1, eA2, eB0, eB1, eB2,
          pb0, pb1, pb2,
          zbuf, zdbuf, sbuf, dbuf, obuf, mv, S, D,
          sem_h, sem_e, sem_d, sem_s, sem_p):
        c = lax.axis_index("c")
        s = lax.axis_index("s")
        iota = lax.iota(jnp.int32, _L)
        zf = jnp.zeros((_L,), jnp.float32)
        bufs = ((hb0, eA0, eB0, pb0, ds0),
                (hb1, eA1, eB1, pb1, ds1),
                (hb2, eA2, eB2, pb2, ds2))

        pltpu.sync_copy(src_hbm.at[s], src_adj)
        pltpu.sync_copy(dst_hbm.at[s], dst_raw)
        pltpu.sync_copy(m_hbm, mv)
        m_vec = mv[...]

        cNv = jnp.full((_L,), c * _N, jnp.int32)
        cMv = jnp.full((_L,), c * _NED, jnp.int32)

        def adj_chunk(j, _):
            def adj_vec(q, _):
                sl = pl.ds(q * _L, _L)
                src_adj[j, sl] = src_adj[j, sl] + cNv
                dst_adj[j, sl] = dst_raw[j, sl] + cMv
                return 0
            return lax.fori_loop(0, _B // _L, adj_vec, 0)
        lax.fori_loop(0, _NCHUNK, adj_chunk, 0)

        def zrow(b, _):
            def zcol(o, _):
                zbuf[b, pl.ds(o * _L, _L)] = zf
                return 0
            return lax.fori_loop(0, R, zcol, 0)
        lax.fori_loop(0, _ZR, zrow, 0)

        iota_div8 = lax.shift_right_arithmetic(iota, 3)
        iota_mod8 = lax.bitwise_and(iota, 7)
        log2Hh = Hh.bit_length() - 1
        iota_divH = lax.shift_right_arithmetic(iota, log2Hh)
        iota_modH = lax.bitwise_and(iota, Hh - 1)

        def zd(g, _):
            r = jnp.full((_L,), g * 2, jnp.int32) + iota_div8
            plsc.store_scatter(zdbuf, [r, iota_mod8], zf)
            return 0
        lax.fori_loop(0, (srpt * _HP) // _L, zd, 0)

        junkv = jnp.full((_L,), junk0, jnp.int32)

        def gather_issue(j, b):
            hb, eA, eB, pbn, dsb = bufs[b]
            sidx = src_adj.at[j]
            pltpu.async_copy(h_hbm.at[sidx], hb, sem_h)
            pltpu.async_copy(es_hbm.at[sidx], eA, sem_e)
            pltpu.async_copy(ed_hbm.at[dst_adj.at[j]], eB, sem_d)

        def scatter_issue(b):
            hb, eA, eB, pbn, dsb = bufs[b]
            pltpu.async_copy(hb, S.at[dsb], sem_s, add=True)
            pltpu.async_copy(pbn, D.at[dsb], sem_p, add=True)

        def scatter_drain(b):
            hb, eA, eB, pbn, dsb = bufs[b]
            pltpu.make_async_copy(hb, S.at[dsb], sem_s).wait()
            pltpu.make_async_copy(pbn, D.at[dsb], sem_p).wait()

        for st in range(n_stripes):
            base = st * stride
            basev = jnp.full((_L,), base, jnp.int32)

            def zs(q, _):
                pltpu.sync_copy(zbuf, S.at[pl.ds(s * srpt + q * _ZR, _ZR)])
                return 0
            lax.fori_loop(0, srpt // _ZR, zs, 0)
            pltpu.sync_copy(zdbuf, D.at[pl.ds(s * srpt, srpt)])

            plsc.subcore_barrier()

            def substep(j, b):
                hb, eA, eB, pbn, dsb = bufs[b]
                nb = (b + 1) % 3
                scatter_drain(nb)
                gather_issue(jnp.minimum(j + 1, _NCHUNK - 1), nb)

                def srow(q, _):
                    sl = pl.ds(q * _L, _L)
                    rel = dst_raw[j, sl] - basev
                    if n_stripes == 1:
                        dsb[sl] = rel
                    else:
                        ok = jnp.logical_and(rel >= 0, rel < stride)
                        junk = junkv + lax.bitwise_and(rel, 255)
                        dsb[sl] = jnp.where(ok, rel, junk)
                    return 0
                lax.fori_loop(0, _B // _L, srow, 0)

                sidx = src_adj.at[j]
                pltpu.make_async_copy(es_hbm.at[sidx], eA, sem_e).wait()
                pltpu.make_async_copy(ed_hbm.at[dst_adj.at[j]], eB, sem_d).wait()

                def pstep(g, _):
                    r = jnp.full((_L,), g * (_L // Hh), jnp.int32) + iota_divH
                    es_v = plsc.load_gather(eA, [r, iota_modH])
                    ed_v = plsc.load_gather(eB, [r, iota_modH])
                    e = es_v + ed_v
                    e = jnp.where(e >= 0.0, e, 0.2 * e)
                    plsc.store_scatter(pbn, [r, iota_modH],
                                       jnp.exp(e - m_vec))
                    return 0
                lax.fori_loop(0, (_B * Hh) // _L, pstep, 0)

                pltpu.make_async_copy(h_hbm.at[sidx], hb, sem_h).wait()

                if Hh == 1:
                    zi = jnp.zeros((_L,), jnp.int32)
                    def mrow(bb, _):
                        bv = jnp.full((_L,), bb, jnp.int32)
                        pv = plsc.load_gather(pbn, [bv, zi])
                        def mcol(o, _):
                            sl = pl.ds(o * _L, _L)
                            hb[bb, sl] = hb[bb, sl] * pv
                            return 0
                        return lax.fori_loop(0, R, mcol, 0)
                else:
                    def mrow(bb, _):
                        bv = jnp.full((_L,), bb, jnp.int32)
                        def mcol(o, _):
                            off = o * _L
                            sl = pl.ds(off, _L)
                            head = lax.shift_right_arithmetic(
                                jnp.full((_L,), off, jnp.int32) + iota, log2C)
                            pv = plsc.load_gather(pbn, [bv, head])
                            hb[bb, sl] = hb[bb, sl] * pv
                            return 0
                        return lax.fori_loop(0, R, mcol, 0)
                lax.fori_loop(0, _B, mrow, 0)

                scatter_issue(b)

            def dfill(b):
                dsb = bufs[b][4]
                def f(q, _):
                    dsb[pl.ds(q * _L, _L)] = junkv
                    return 0
                lax.fori_loop(0, _B // _L, f, 0)
            dfill(1)
            dfill(2)
            scatter_issue(1)
            scatter_issue(2)
            gather_issue(jnp.int32(0), 0)

            def tri(g, _):
                j = g * 3
                substep(j, 0)
                substep(j + 1, 1)
                substep(j + 2, 2)
                return 0
            lax.fori_loop(0, _NCHUNK // 3, tri, 0)

            scatter_drain(1)
            scatter_drain(2)
            hb, eA, eB, pbn, dsb = bufs[0]
            jl = jnp.int32(_NCHUNK - 1)
            pltpu.make_async_copy(es_hbm.at[src_adj.at[jl]], eA, sem_e).wait()
            pltpu.make_async_copy(ed_hbm.at[dst_adj.at[jl]], eB, sem_d).wait()
            pltpu.make_async_copy(h_hbm.at[src_adj.at[jl]], hb, sem_h).wait()

            plsc.subcore_barrier()

            def rd(q, _):
                r0 = s * rpt + q * _ZR
                pltpu.sync_copy(S.at[pl.ds(r0, _ZR)], sbuf)
                pltpu.sync_copy(D.at[pl.ds(r0, _ZR)], dbuf)

                if Hh == 1:
                    zi = jnp.zeros((_L,), jnp.int32)
                    def rrow(b, _):
                        bv = jnp.full((_L,), b, jnp.int32)
                        dv = plsc.load_gather(dbuf, [bv, zi]) + 1e-16
                        def rcol(o, _):
                            sl = pl.ds(o * _L, _L)
                            obuf[b, sl] = sbuf[b, sl] / dv
                            return 0
                        return lax.fori_loop(0, R, rcol, 0)
                else:
                    def rrow(b, _):
                        bv = jnp.full((_L,), b, jnp.int32)
                        def rcol(o, _):
                            off = o * _L
                            sl = pl.ds(off, _L)
                            head = lax.shift_right_arithmetic(
                                jnp.full((_L,), off, jnp.int32) + iota, log2C)
                            dv = plsc.load_gather(dbuf, [bv, head])
                            obuf[b, sl] = sbuf[b, sl] / (dv + 1e-16)
                            return 0
                        return lax.fori_loop(0, R, rcol, 0)
                lax.fori_loop(0, _ZR, rrow, 0)

                pltpu.sync_copy(obuf, out_hbm.at[c].at[pl.ds(base + r0, _ZR)])
                return 0
            lax.fori_loop(0, rpt // _ZR, rd, 0)

            if st + 1 < n_stripes:
                plsc.subcore_barrier()

    return k


_edge_sc1 = _make_edge_sc(_H1 // 2, _C1, (_H1 * _C1) // 2, 1)
_edge_sc2 = _make_edge_sc(1, _OUT, _OUT // 2, 2)


def _split_cols(t, Fh):
    n = t.shape[0]
    return t.reshape(n, 2, Fh).transpose(1, 0, 2).reshape(2 * n, Fh)


def _pad_heads(t):
    return jnp.pad(t, ((0, 0), (0, _HP - t.shape[1])))


def _shift_upper_bound(es, ed):
    m = jnp.max(es) + jnp.max(ed)
    m = jnp.where(m >= 0.0, m, 0.2 * m)
    return jnp.full((16,), m, jnp.float32)


def kernel(x, edge_index, W1, att_src1, att_dst1, b1, W2, att_src2, att_dst2, b2):
    loop = jnp.arange(_N, dtype=jnp.int32)
    pad = _ETP - (_E + _N)
    src = jnp.concatenate(
        [edge_index[0].astype(jnp.int32), loop, jnp.zeros((pad,), jnp.int32)])
    dst = jnp.concatenate(
        [edge_index[1].astype(jnp.int32), loop,
         jnp.full((pad,), _N, jnp.int32)])
    src2d = src.reshape(_NS, _NCHUNK, _B)
    dst2d = dst.reshape(_NS, _NCHUNK, _B)

    h1, es1, ed1 = _dense_stage(x, W1, att_src1, att_dst1, _H1, _C1)
    m1 = _shift_upper_bound(es1, ed1)
    h1s = _split_cols(h1, (_H1 * _C1) // 2)
    es1s = _pad_heads(_split_cols(es1, _H1 // 2))
    ed1p = jnp.concatenate([ed1, jnp.zeros((_NED - _N, _H1), jnp.float32)], 0)
    ed1s = _pad_heads(_split_cols(ed1p, _H1 // 2))
    agg1 = _edge_sc1(src2d, dst2d, h1s, es1s, ed1s, m1)
    agg1 = jnp.concatenate([agg1[0, :_N], agg1[1, :_N]], axis=1)

    h2, es2, ed2 = _dense_stage(agg1, W2, att_src2, att_dst2, 1, _OUT,
                                bias=b1)
    m2 = _shift_upper_bound(es2, ed2)
    h2s = _split_cols(h2, _OUT // 2)
    es2s = _pad_heads(jnp.concatenate([es2, es2], 0))
    ed2p = jnp.concatenate([ed2, jnp.zeros((_NED - _N, 1), jnp.float32)], 0)
    ed2s = _pad_heads(jnp.concatenate([ed2p, ed2p], 0))
    agg2 = _edge_sc2(src2d, dst2d, h2s, es2s, ed2s, m2)
    agg2 = jnp.concatenate([agg2[0, :_N], agg2[1, :_N]], axis=1)

    return _log_softmax_bias(agg2, b2)


# speedup vs baseline: 45.0445x; 1.0016x over previous
import functools

import jax
import jax.numpy as jnp
from jax import lax
from jax.experimental import pallas as pl
from jax.experimental.pallas import tpu as pltpu
from jax.experimental.pallas import tpu_sc as plsc

_N = 10000
_E = 320000
_IN = 128
_OUT = 128
_H1 = 8
_C1 = 8

_ROWS = 1000

_NC, _NS, _L = 2, 16, 16
_B = 128
_NCHUNK = 162
_ETP = _NS * _NCHUNK * _B
_NP = 10240
_RPT = _NP // _NS
_ZR = 32
_NED = _N + 8
_HP = 8


def _dense1_body(x_ref, w_ref, asrc_ref, adst_ref, h_ref, es_ref, ed_ref):
    h = jnp.dot(x_ref[...], w_ref[...], preferred_element_type=jnp.float32)
    h_ref[...] = h
    es_ref[...] = jnp.dot(h, asrc_ref[...], preferred_element_type=jnp.float32)
    ed_ref[...] = jnp.dot(h, adst_ref[...], preferred_element_type=jnp.float32)


def _dense2_body(x_ref, b_ref, w_ref, asrc_ref, adst_ref, h_ref, es_ref, ed_ref):
    o = jnp.maximum(x_ref[...] + b_ref[...], 0.0)
    h = jnp.dot(o, w_ref[...], preferred_element_type=jnp.float32)
    h_ref[...] = h
    es_ref[...] = jnp.dot(h, asrc_ref[...], preferred_element_type=jnp.float32)
    ed_ref[...] = jnp.dot(h, adst_ref[...], preferred_element_type=jnp.float32)


def _att_proj(a, heads, ch):
    eye = jnp.eye(heads, dtype=jnp.float32)
    return (a.reshape(heads, ch)[:, :, None] * eye[:, None, :]).reshape(
        heads * ch, heads)


def _dense_stage(x, W, a_src, a_dst, heads, ch, bias=None):
    n, k = x.shape[0], W.shape[1]
    Asrc = _att_proj(a_src, heads, ch)
    Adst = _att_proj(a_dst, heads, ch)
    grid = n // _ROWS
    in_specs = [
        pl.BlockSpec((_ROWS, x.shape[1]), lambda i: (i, 0)),
        pl.BlockSpec((x.shape[1], k), lambda i: (0, 0)),
        pl.BlockSpec((k, heads), lambda i: (0, 0)),
        pl.BlockSpec((k, heads), lambda i: (0, 0)),
    ]
    args = (x, W, Asrc, Adst)
    body = _dense1_body
    if bias is not None:
        in_specs.insert(1, pl.BlockSpec((1, x.shape[1]), lambda i: (0, 0)))
        args = (x, bias.reshape(1, -1), W, Asrc, Adst)
        body = _dense2_body
    return pl.pallas_call(
        body,
        grid=(grid,),
        in_specs=in_specs,
        out_specs=[
            pl.BlockSpec((_ROWS, k), lambda i: (i, 0)),
            pl.BlockSpec((_ROWS, heads), lambda i: (i, 0)),
            pl.BlockSpec((_ROWS, heads), lambda i: (i, 0)),
        ],
        out_shape=[
            jax.ShapeDtypeStruct((n, k), jnp.float32),
            jax.ShapeDtypeStruct((n, heads), jnp.float32),
            jax.ShapeDtypeStruct((n, heads), jnp.float32),
        ],
    )(*args)


def _lsm_body(x_ref, b_ref, o_ref):
    z = x_ref[...] + b_ref[...]
    m = jnp.max(z, axis=1, keepdims=True)
    ez = jnp.exp(z - m)
    lse = jnp.log(jnp.sum(ez, axis=1, keepdims=True))
    o_ref[...] = z - m - lse


def _log_softmax_bias(x, b):
    n, k = x.shape
    return pl.pallas_call(
        _lsm_body,
        grid=(n // _ROWS,),
        in_specs=[
            pl.BlockSpec((_ROWS, k), lambda i: (i, 0)),
            pl.BlockSpec((1, k), lambda i: (0, 0)),
        ],
        out_specs=pl.BlockSpec((_ROWS, k), lambda i: (i, 0)),
        out_shape=jax.ShapeDtypeStruct((n, k), jnp.float32),
    )(x, b.reshape(1, k))


def _make_edge_sc(Hh, C, Fh, n_stripes):
    R = Fh // _L
    log2C = C.bit_length() - 1
    stride = _NP // n_stripes
    srows = _NP if n_stripes == 1 else 5632
    srpt = srows // _NS
    rpt = stride // _NS
    junk0 = stride if n_stripes > 1 else _N + 8
    mesh = plsc.VectorSubcoreMesh(core_axis_name="c", subcore_axis_name="s")

    @functools.partial(
        pl.kernel,
        out_type=jax.ShapeDtypeStruct((2, _NP, Fh), jnp.float32),
        mesh=mesh,
        compiler_params=pltpu.CompilerParams(
            needs_layout_passes=False, use_tc_tiling_on_sc=False),
        scratch_types=[
            pltpu.VMEM((_NCHUNK, _B), jnp.int32),
            pltpu.VMEM((_NCHUNK, _B), jnp.int32),
            pltpu.VMEM((_NCHUNK, _B), jnp.int32),
            pltpu.VMEM((_B,), jnp.int32),
            pltpu.VMEM((_B,), jnp.int32),
            pltpu.VMEM((_B,), jnp.int32),
            pltpu.VMEM((_B, Fh), jnp.float32),
            pltpu.VMEM((_B, Fh), jnp.float32),
            pltpu.VMEM((_B, Fh), jnp.float32),
            pltpu.VMEM((_B, _HP), jnp.float32),
            pltpu.VMEM((_B, _HP), jnp.float32),
            pltpu.VMEM((_B, _HP), jnp.float32),
            pltpu.VMEM((_B, _HP), jnp.float32),
            pltpu.VMEM((_B, _HP), jnp.float32),
            pltpu.VMEM((_B, _HP), jnp.float32),
            pltpu.VMEM((_B, _HP), jnp.float32),
            pltpu.VMEM((_B, _HP), jnp.float32),
            pltpu.VMEM((_B, _HP), jnp.float32),
            pltpu.VMEM((_ZR, Fh), jnp.float32),
            pltpu.VMEM((srpt, _HP), jnp.float32),
            pltpu.VMEM((_ZR, Fh), jnp.float32),
            pltpu.VMEM((_ZR, _HP), jnp.float32),
            pltpu.VMEM((_ZR, Fh), jnp.float32),
            pltpu.VMEM((16,), jnp.float32),
            pltpu.VMEM_SHARED((srows, Fh), jnp.float32),
            pltpu.VMEM_SHARED((srows, _HP), jnp.float32),
            pltpu.SemaphoreType.DMA,
            pltpu.SemaphoreType.DMA,
            pltpu.SemaphoreType.DMA,
            pltpu.SemaphoreType.DMA,
            pltpu.SemaphoreType.DMA,
        ],
    )
    def k(src_hbm, dst_hbm, h_hbm, es_hbm, ed_hbm, m_hbm, out_hbm,
          src_adj, dst_raw, dst_adj,
          ds0, ds1, ds2, hb0, hb1, hb2, eA0, eA1, eA2, eB0, eB1, eB2,
          pb0, pb1, pb2,
          zbuf, zdbuf, sbuf, dbuf, obuf, mv, S, D,
          sem_h, sem_e, sem_d, sem_s, sem_p):
        c = lax.axis_index("c")
        s = lax.axis_index("s")
        iota = lax.iota(jnp.int32, _L)
        zf = jnp.zeros((_L,), jnp.float32)
        bufs = ((hb0, eA0, eB0, pb0, ds0),
                (hb1, eA1, eB1, pb1, ds1),
                (hb2, eA2, eB2, pb2, ds2))

        pltpu.sync_copy(src_hbm.at[s], src_adj)
        pltpu.sync_copy(dst_hbm.at[s], dst_raw)
        pltpu.sync_copy(m_hbm, mv)
        m_vec = mv[...]

        cNv = jnp.full((_L,), c * _N, jnp.int32)
        cMv = jnp.full((_L,), c * _NED, jnp.int32)

        def adj_chunk(j, _):
            def adj_vec(q, _):
                sl = pl.ds(q * _L, _L)
                src_adj[j, sl] = src_adj[j, sl] + cNv
                dst_adj[j, sl] = dst_raw[j, sl] + cMv
                return 0
            return lax.fori_loop(0, _B // _L, adj_vec, 0)
        lax.fori_loop(0, _NCHUNK, adj_chunk, 0)

        def zrow(b, _):
            def zcol(o, _):
                zbuf[b, pl.ds(o * _L, _L)] = zf
                return 0
            return lax.fori_loop(0, R, zcol, 0)
        lax.fori_loop(0, _ZR, zrow, 0)

        iota_div8 = lax.shift_right_arithmetic(iota, 3)
        iota_mod8 = lax.bitwise_and(iota, 7)
        log2Hh = Hh.bit_length() - 1
        iota_divH = lax.shift_right_arithmetic(iota, log2Hh)
        iota_modH = lax.bitwise_and(iota, Hh - 1)

        def zd(g, _):
            r = jnp.full((_L,), g * 2, jnp.int32) + iota_div8
            plsc.store_scatter(zdbuf, [r, iota_mod8], zf)
            return 0
        lax.fori_loop(0, (srpt * _HP) // _L, zd, 0)

        junkv = jnp.full((_L,), junk0, jnp.int32)

        def gather_issue(j, b):
            hb, eA, eB, pbn, dsb = bufs[b]
            sidx = src_adj.at[j]
            pltpu.async_copy(h_hbm.at[sidx], hb, sem_h)
            pltpu.async_copy(es_hbm.at[sidx], eA, sem_e)
            pltpu.async_copy(ed_hbm.at[dst_adj.at[j]], eB, sem_d)

        def scatter_issue(b):
            hb, eA, eB, pbn, dsb = bufs[b]
            pltpu.async_copy(hb, S.at[dsb], sem_s, add=True)
            pltpu.async_copy(pbn, D.at[dsb], sem_p, add=True)

        def scatter_drain(b):
            hb, eA, eB, pbn, dsb = bufs[b]
            pltpu.make_async_copy(hb, S.at[dsb], sem_s).wait()
            pltpu.make_async_copy(pbn, D.at[dsb], sem_p).wait()

        for st in range(n_stripes):
            base = st * stride
            basev = jnp.full((_L,), base, jnp.int32)

            def zs(q, _):
                pltpu.sync_copy(zbuf, S.at[pl.ds(s * srpt + q * _ZR, _ZR)])
                return 0
            lax.fori_loop(0, srpt // _ZR, zs, 0)
            pltpu.sync_copy(zdbuf, D.at[pl.ds(s * srpt, srpt)])

            plsc.subcore_barrier()

            def substep(j, b):
                hb, eA, eB, pbn, dsb = bufs[b]
                nb = (b + 1) % 3
                scatter_drain(nb)
                gather_issue(jnp.minimum(j + 1, _NCHUNK - 1), nb)

                def srow(q, _):
                    sl = pl.ds(q * _L, _L)
                    rel = dst_raw[j, sl] - basev
                    if n_stripes == 1:
                        dsb[sl] = rel
                    else:
                        ok = jnp.logical_and(rel >= 0, rel < stride)
                        junk = junkv + lax.bitwise_and(rel, 255)
                        dsb[sl] = jnp.where(ok, rel, junk)
                    return 0
                lax.fori_loop(0, _B // _L, srow, 0)

                sidx = src_adj.at[j]
                pltpu.make_async_copy(es_hbm.at[sidx], eA, sem_e).wait()
                pltpu.make_async_copy(ed_hbm.at[dst_adj.at[j]], eB, sem_d).wait()

                def pstep(g, _):
                    r = jnp.full((_L,), g * (_L // Hh), jnp.int32) + iota_divH
                    es_v = plsc.load_gather(eA, [r, iota_modH])
                    ed_v = plsc.load_gather(eB, [r, iota_modH])
                    e = es_v + ed_v
                    e = jnp.where(e >= 0.0, e, 0.2 * e)
                    plsc.store_scatter(pbn, [r, iota_modH],
                                       jnp.exp(e - m_vec))
                    return 0
                lax.fori_loop(0, (_B * Hh) // _L, pstep, 0)

                pltpu.make_async_copy(h_hbm.at[sidx], hb, sem_h).wait()

                if Hh == 1:
                    zi = jnp.zeros((_L,), jnp.int32)
                    def mrow(bb, _):
                        bv = jnp.full((_L,), bb, jnp.int32)
                        pv = plsc.load_gather(pbn, [bv, zi])
                        def mcol(o, _):
                            sl = pl.ds(o * _L, _L)
                            hb[bb, sl] = hb[bb, sl] * pv
                            return 0
                        return lax.fori_loop(0, R, mcol, 0)
                else:
                    def mrow(bb, _):
                        bv = jnp.full((_L,), bb, jnp.int32)
                        def mcol(o, _):
                            off = o * _L
                            sl = pl.ds(off, _L)
                            head = lax.shift_right_arithmetic(
                                jnp.full((_L,), off, jnp.int32) + iota, log2C)
                            pv = plsc.load_gather(pbn, [bv, head])
                            hb[bb, sl] = hb[bb, sl] * pv
                            return 0
                        return lax.fori_loop(0, R, mcol, 0)
                lax.fori_loop(0, _B, mrow, 0)

                scatter_issue(b)

            def dfill(b):
                dsb = bufs[b][4]
                def f(q, _):
                    dsb[pl.ds(q * _L, _L)] = junkv
                    return 0
                lax.fori_loop(0, _B // _L, f, 0)
            dfill(1)
            dfill(2)
            scatter_issue(1)
            scatter_issue(2)
            gather_issue(jnp.int32(0), 0)

            def tri(g, _):
                j = g * 3
                substep(j, 0)
                substep(j + 1, 1)
                substep(j + 2, 2)
                return 0
            lax.fori_loop(0, _NCHUNK // 3, tri, 0)

            scatter_drain(1)
            scatter_drain(2)
            hb, eA, eB, pbn, dsb = bufs[0]
            jl = jnp.int32(_NCHUNK - 1)
            pltpu.make_async_copy(es_hbm.at[src_adj.at[jl]], eA, sem_e).wait()
            pltpu.make_async_copy(ed_hbm.at[dst_adj.at[jl]], eB, sem_d).wait()
            pltpu.make_async_copy(h_hbm.at[src_adj.at[jl]], hb, sem_h).wait()

            plsc.subcore_barrier()

            def rd(q, _):
                r0 = s * rpt + q * _ZR
                pltpu.sync_copy(S.at[pl.ds(r0, _ZR)], sbuf)
                pltpu.sync_copy(D.at[pl.ds(r0, _ZR)], dbuf)

                if Hh == 1:
                    zi = jnp.zeros((_L,), jnp.int32)
                    def rrow(b, _):
                        bv = jnp.full((_L,), b, jnp.int32)
                        dv = plsc.load_gather(dbuf, [bv, zi]) + 1e-16
                        def rcol(o, _):
                            sl = pl.ds(o * _L, _L)
                            obuf[b, sl] = sbuf[b, sl] / dv
                            return 0
                        return lax.fori_loop(0, R, rcol, 0)
                else:
                    def rrow(b, _):
                        bv = jnp.full((_L,), b, jnp.int32)
                        def rcol(o, _):
                            off = o * _L
                            sl = pl.ds(off, _L)
                            head = lax.shift_right_arithmetic(
                                jnp.full((_L,), off, jnp.int32) + iota, log2C)
                            dv = plsc.load_gather(dbuf, [bv, head])
                            obuf[b, sl] = sbuf[b, sl] / (dv + 1e-16)
                            return 0
                        return lax.fori_loop(0, R, rcol, 0)
                lax.fori_loop(0, _ZR, rrow, 0)

                pltpu.sync_copy(obuf, out_hbm.at[c].at[pl.ds(base + r0, _ZR)])
                return 0
            lax.fori_loop(0, rpt // _ZR, rd, 0)

            if st + 1 < n_stripes:
                plsc.subcore_barrier()

    return k


@functools.lru_cache(maxsize=None)
def _edge_sc_cached(Hh, C, Fh, n_stripes):
    return _make_edge_sc(Hh, C, Fh, n_stripes)


def _split_cols(t, Fh):
    n = t.shape[0]
    return t.reshape(n, 2, Fh).transpose(1, 0, 2).reshape(2 * n, Fh)


def _pad_heads(t):
    return jnp.pad(t, ((0, 0), (0, _HP - t.shape[1])))


def _shift_upper_bound(es, ed):
    m = jnp.max(es) + jnp.max(ed)
    m = jnp.where(m >= 0.0, m, 0.2 * m)
    return jnp.full((16,), m, jnp.float32)


def kernel(x, edge_index, W1, att_src1, att_dst1, b1, W2, att_src2, att_dst2, b2):
    loop = jnp.arange(_N, dtype=jnp.int32)
    pad = _ETP - (_E + _N)
    src = jnp.concatenate(
        [edge_index[0].astype(jnp.int32), loop, jnp.zeros((pad,), jnp.int32)])
    dst = jnp.concatenate(
        [edge_index[1].astype(jnp.int32), loop,
         jnp.full((pad,), _N, jnp.int32)])
    src2d = src.reshape(_NS, _NCHUNK, _B)
    dst2d = dst.reshape(_NS, _NCHUNK, _B)

    h1, es1, ed1 = _dense_stage(x, W1, att_src1, att_dst1, _H1, _C1)
    m1 = _shift_upper_bound(es1, ed1)
    h1s = _split_cols(h1, (_H1 * _C1) // 2)
    es1s = _pad_heads(_split_cols(es1, _H1 // 2))
    ed1p = jnp.concatenate([ed1, jnp.zeros((_NED - _N, _H1), jnp.float32)], 0)
    ed1s = _pad_heads(_split_cols(ed1p, _H1 // 2))
    _edge_sc1 = _edge_sc_cached(_H1 // 2, _C1, (_H1 * _C1) // 2, 1)
    agg1 = _edge_sc1(src2d, dst2d, h1s, es1s, ed1s, m1)
    agg1 = jnp.concatenate([agg1[0, :_N], agg1[1, :_N]], axis=1)

    h2, es2, ed2 = _dense_stage(agg1, W2, att_src2, att_dst2, 1, _OUT,
                                bias=b1)
    m2 = _shift_upper_bound(es2, ed2)
    h2s = _split_cols(h2, _OUT // 2)
    es2s = _pad_heads(jnp.concatenate([es2, es2], 0))
    ed2p = jnp.concatenate([ed2, jnp.zeros((_NED - _N, 1), jnp.float32)], 0)
    ed2s = _pad_heads(jnp.concatenate([ed2p, ed2p], 0))
    _edge_sc2 = _edge_sc_cached(1, _OUT, _OUT // 2, 2)
    agg2 = _edge_sc2(src2d, dst2d, h2s, es2s, ed2s, m2)
    agg2 = jnp.concatenate([agg2[0, :_N], agg2[1, :_N]], axis=1)

    return _log_softmax_bias(agg2, b2)
